# baseline (speedup 1.0000x reference)
import functools

import jax
import jax.numpy as jnp
from jax import lax
from jax.experimental import pallas as pl
from jax.experimental.pallas import tpu as pltpu
from jax.experimental.pallas import tpu_sc as plsc

EPS = 1e-12

NC = 2
NS = 16
NW = NC * NS


def _sc_gather(table, ids_flat, chunk):
    n, h = ids_flat.shape[0], table.shape[1]
    rows_per_w = n // NW
    nchunk = rows_per_w // chunk
    mesh = plsc.VectorSubcoreMesh(
        core_axis_name="c", subcore_axis_name="s", num_cores=NC, num_subcores=NS
    )

    @functools.partial(
        pl.kernel,
        mesh=mesh,
        out_type=jax.ShapeDtypeStruct((n, h), jnp.float32),
        scratch_types=[
            pltpu.VMEM((rows_per_w,), jnp.int32),
            pltpu.VMEM((2, chunk, h), jnp.float32),
            pltpu.SemaphoreType.DMA((2,)),
            pltpu.SemaphoreType.DMA((2,)),
        ],
    )
    def gather_kernel(ids_hbm, table_hbm, out_hbm, idx_v, rows_v, gsem, wsem):
        wid = lax.axis_index("s") * NC + lax.axis_index("c")
        base = wid * rows_per_w
        pltpu.sync_copy(ids_hbm.at[pl.ds(base, rows_per_w)], idx_v)

        def start_gather(c):
            pltpu.async_copy(
                table_hbm.at[idx_v.at[pl.ds(c * chunk, chunk)]],
                rows_v.at[c % 2],
                gsem.at[c % 2],
            )

        def wait_gather(c):
            pltpu.make_async_copy(
                table_hbm.at[idx_v.at[pl.ds(c * chunk, chunk)]],
                rows_v.at[c % 2],
                gsem.at[c % 2],
            ).wait()

        def start_writeback(c):
            pltpu.async_copy(
                rows_v.at[c % 2],
                out_hbm.at[pl.ds(base + c * chunk, chunk)],
                wsem.at[c % 2],
            )

        def wait_writeback(c):
            pltpu.make_async_copy(
                rows_v.at[c % 2],
                out_hbm.at[pl.ds(base + c * chunk, chunk)],
                wsem.at[c % 2],
            ).wait()

        start_gather(0)
        for c in range(nchunk):
            wait_gather(c)
            if c + 1 < nchunk:
                if c >= 1:
                    wait_writeback(c - 1)
                start_gather(c + 1)
            start_writeback(c)
        if nchunk >= 2:
            wait_writeback(nchunk - 2)
        wait_writeback(nchunk - 1)

    return gather_kernel(ids_flat, table)


def _tc_fuse_slice(gathered, ttf, pos_emb, type_emb, gamma2, beta2, out_prev, k, br, n):
    seq, h = gathered.shape
    nblk = seq // br

    def body(g_ref, tt_ref, pos_ref, type_ref, gam_ref, bet_ref, *refs):
        o_ref = refs[-1]
        x = g_ref[...] + pos_ref[...]
        ttv = tt_ref[...].astype(jnp.float32)
        t0 = type_ref[0:1, :]
        t1 = type_ref[1:2, :]
        x = x + t0 + ttv * (t1 - t0)
        mean = jnp.mean(x, axis=-1, keepdims=True)
        xc = x - mean
        var = jnp.mean(xc * xc, axis=-1, keepdims=True)
        inv = lax.rsqrt(var + EPS)
        o_ref[...] = xc * inv * gam_ref[...] + bet_ref[...]

    in_specs = [
        pl.BlockSpec((br, h), lambda p: (p, 0)),
        pl.BlockSpec((br, 1), lambda p: (p, 0)),
        pl.BlockSpec((br, h), lambda p: (p, 0)),
        pl.BlockSpec((2, h), lambda p: (0, 0)),
        pl.BlockSpec((1, h), lambda p: (0, 0)),
        pl.BlockSpec((1, h), lambda p: (0, 0)),
    ]
    args = [gathered, ttf, pos_emb, type_emb, gamma2, beta2]
    aliases = {}
    if out_prev is not None:
        in_specs.append(pl.BlockSpec(memory_space=pl.ANY))
        args.append(out_prev)
        aliases = {6: 0}
    return pl.pallas_call(
        body,
        grid=(nblk,),
        in_specs=in_specs,
        out_specs=pl.BlockSpec((br, h), lambda p, _k=k, _nblk=nblk: (_k * _nblk + p, 0)),
        out_shape=jax.ShapeDtypeStruct((n, h), jnp.float32),
        input_output_aliases=aliases,
    )(*args)


def kernel(input_ids, token_type_ids, word_emb, pos_emb, type_emb, gamma, beta):
    b, s = input_ids.shape
    h = word_emb.shape[1]
    n = b * s
    ids_flat = input_ids.reshape(n).astype(jnp.int32)
    ttf = token_type_ids.reshape(n, 1).astype(jnp.int32)
    gamma2 = gamma.reshape(1, h)
    beta2 = beta.reshape(1, h)
    gathers = [
        _sc_gather(word_emb, ids_flat[k * s : (k + 1) * s], chunk=32)
        for k in range(b)
    ]
    out = None
    for k in range(b):
        out = _tc_fuse_slice(
            gathers[k],
            ttf[k * s : (k + 1) * s],
            pos_emb,
            type_emb,
            gamma2,
            beta2,
            out,
            k,
            br=2048,
            n=n,
        )
    return out.reshape(b, s, h)


# speedup vs baseline: 1.1560x; 1.1560x over previous
import functools

import jax
import jax.numpy as jnp
from jax import lax
from jax.experimental import pallas as pl
from jax.experimental.pallas import tpu as pltpu
from jax.experimental.pallas import tpu_sc as plsc

EPS = 1e-12

NC = 2
NS = 16
NW = NC * NS


def _sc_gather(table, ids_flat, chunk):
    n, h = ids_flat.shape[0], table.shape[1]
    rows_per_w = n // NW
    nchunk = rows_per_w // chunk
    mesh = plsc.VectorSubcoreMesh(
        core_axis_name="c", subcore_axis_name="s", num_cores=NC, num_subcores=NS
    )

    @functools.partial(
        pl.kernel,
        mesh=mesh,
        out_type=jax.ShapeDtypeStruct((n, h), jnp.float32),
        scratch_types=[
            pltpu.VMEM((rows_per_w,), jnp.int32),
            pltpu.VMEM((2, chunk, h), jnp.float32),
            pltpu.SemaphoreType.DMA((2,)),
            pltpu.SemaphoreType.DMA((2,)),
        ],
    )
    def gather_kernel(ids_hbm, table_hbm, out_hbm, idx_v, rows_v, gsem, wsem):
        wid = lax.axis_index("s") * NC + lax.axis_index("c")
        base = wid * rows_per_w
        pltpu.sync_copy(ids_hbm.at[pl.ds(base, rows_per_w)], idx_v)

        def start_gather(c):
            pltpu.async_copy(
                table_hbm.at[idx_v.at[pl.ds(c * chunk, chunk)]],
                rows_v.at[c % 2],
                gsem.at[c % 2],
            )

        def wait_gather(c):
            pltpu.make_async_copy(
                table_hbm.at[idx_v.at[pl.ds(c * chunk, chunk)]],
                rows_v.at[c % 2],
                gsem.at[c % 2],
            ).wait()

        def start_writeback(c):
            pltpu.async_copy(
                rows_v.at[c % 2],
                out_hbm.at[pl.ds(base + c * chunk, chunk)],
                wsem.at[c % 2],
            )

        def wait_writeback(c):
            pltpu.make_async_copy(
                rows_v.at[c % 2],
                out_hbm.at[pl.ds(base + c * chunk, chunk)],
                wsem.at[c % 2],
            ).wait()

        start_gather(0)
        for c in range(nchunk):
            wait_gather(c)
            if c + 1 < nchunk:
                if c >= 1:
                    wait_writeback(c - 1)
                start_gather(c + 1)
            start_writeback(c)
        if nchunk >= 2:
            wait_writeback(nchunk - 2)
        wait_writeback(nchunk - 1)

    return gather_kernel(ids_flat, table)


def _tc_fuse_slice(gathered, ttf, pos_emb, type_emb, gamma2, beta2, out_prev, k, br, n):
    seq, h = gathered.shape
    nblk = seq // br

    def body(g_ref, tt_ref, pos_ref, type_ref, gam_ref, bet_ref, *refs):
        o_ref = refs[-1]
        x = g_ref[...] + pos_ref[...]
        ttv = tt_ref[...].astype(jnp.float32)
        t0 = type_ref[0:1, :]
        t1 = type_ref[1:2, :]
        x = x + t0 + ttv * (t1 - t0)
        mean = jnp.mean(x, axis=-1, keepdims=True)
        xc = x - mean
        var = jnp.mean(xc * xc, axis=-1, keepdims=True)
        inv = lax.rsqrt(var + EPS)
        o_ref[...] = xc * inv * gam_ref[...] + bet_ref[...]

    in_specs = [
        pl.BlockSpec((br, h), lambda p: (p, 0)),
        pl.BlockSpec((br, 1), lambda p: (p, 0)),
        pl.BlockSpec((br, h), lambda p: (0, 0)),
        pl.BlockSpec((2, h), lambda p: (0, 0)),
        pl.BlockSpec((1, h), lambda p: (0, 0)),
        pl.BlockSpec((1, h), lambda p: (0, 0)),
    ]
    args = [gathered, ttf, pos_emb, type_emb, gamma2, beta2]
    aliases = {}
    if out_prev is not None:
        in_specs.append(pl.BlockSpec(memory_space=pl.ANY))
        args.append(out_prev)
        aliases = {6: 0}
    return pl.pallas_call(
        body,
        grid=(nblk,),
        in_specs=in_specs,
        out_specs=pl.BlockSpec((br, h), lambda p, _k=k, _nblk=nblk: (_k * _nblk + p, 0)),
        out_shape=jax.ShapeDtypeStruct((n, h), jnp.float32),
        input_output_aliases=aliases,
    )(*args)


def kernel(input_ids, token_type_ids, word_emb, pos_emb, type_emb, gamma, beta):
    b, s = input_ids.shape
    h = word_emb.shape[1]
    n = b * s
    ids_flat = input_ids.reshape(n).astype(jnp.int32)
    ttf = token_type_ids.reshape(n, 1).astype(jnp.int32)
    gamma2 = gamma.reshape(1, h)
    beta2 = beta.reshape(1, h)
    nslice = 2
    sb = b // nslice
    sl = sb * s
    gathers = [
        _sc_gather(word_emb, ids_flat[k * sl : (k + 1) * sl], chunk=32)
        for k in range(nslice)
    ]
    out = None
    for k in range(nslice):
        out = _tc_fuse_slice(
            gathers[k],
            ttf[k * sl : (k + 1) * sl],
            pos_emb,
            type_emb,
            gamma2,
            beta2,
            out,
            k,
            br=2048,
            n=n,
        )
    return out.reshape(b, s, h)


# speedup vs baseline: 1.1924x; 1.0315x over previous
import functools

import jax
import jax.numpy as jnp
from jax import lax
from jax.experimental import pallas as pl
from jax.experimental.pallas import tpu as pltpu
from jax.experimental.pallas import tpu_sc as plsc

EPS = 1e-12

NC = 2
NS = 16
NW = NC * NS


def _sc_gather(table, ids_flat, chunk):
    n, h = ids_flat.shape[0], table.shape[1]
    rows_per_w = n // NW
    nchunk = rows_per_w // chunk
    mesh = plsc.VectorSubcoreMesh(
        core_axis_name="c", subcore_axis_name="s", num_cores=NC, num_subcores=NS
    )

    @functools.partial(
        pl.kernel,
        mesh=mesh,
        out_type=jax.ShapeDtypeStruct((n, h), jnp.float32),
        scratch_types=[
            pltpu.VMEM((rows_per_w,), jnp.int32),
            pltpu.VMEM((2, chunk, h), jnp.float32),
            pltpu.SemaphoreType.DMA((2,)),
            pltpu.SemaphoreType.DMA((2,)),
        ],
    )
    def gather_kernel(ids_hbm, table_hbm, out_hbm, idx_v, rows_v, gsem, wsem):
        wid = lax.axis_index("s") * NC + lax.axis_index("c")
        base = wid * rows_per_w
        pltpu.sync_copy(ids_hbm.at[pl.ds(base, rows_per_w)], idx_v)

        def start_gather(c):
            pltpu.async_copy(
                table_hbm.at[idx_v.at[pl.ds(c * chunk, chunk)]],
                rows_v.at[c % 2],
                gsem.at[c % 2],
            )

        def wait_gather(c):
            pltpu.make_async_copy(
                table_hbm.at[idx_v.at[pl.ds(c * chunk, chunk)]],
                rows_v.at[c % 2],
                gsem.at[c % 2],
            ).wait()

        def start_writeback(c):
            pltpu.async_copy(
                rows_v.at[c % 2],
                out_hbm.at[pl.ds(base + c * chunk, chunk)],
                wsem.at[c % 2],
            )

        def wait_writeback(c):
            pltpu.make_async_copy(
                rows_v.at[c % 2],
                out_hbm.at[pl.ds(base + c * chunk, chunk)],
                wsem.at[c % 2],
            ).wait()

        start_gather(0)
        for c in range(nchunk):
            wait_gather(c)
            if c + 1 < nchunk:
                if c >= 1:
                    wait_writeback(c - 1)
                start_gather(c + 1)
            start_writeback(c)
        if nchunk >= 2:
            wait_writeback(nchunk - 2)
        wait_writeback(nchunk - 1)

    return gather_kernel(ids_flat, table)


def _tc_fuse_slice(gathered, ttf, pos_emb, type_emb, gamma2, beta2, out_prev, k, br, n):
    seq, h = gathered.shape
    nblk = seq // br

    def body(g_ref, tt_ref, pos_ref, type_ref, gam_ref, bet_ref, *refs):
        o_ref = refs[-1]
        x = g_ref[...] + pos_ref[...]
        ttv = tt_ref[...].astype(jnp.float32)
        t0 = type_ref[0:1, :]
        t1 = type_ref[1:2, :]
        x = x + t0 + ttv * (t1 - t0)
        mean = jnp.mean(x, axis=-1, keepdims=True)
        xc = x - mean
        var = jnp.mean(xc * xc, axis=-1, keepdims=True)
        inv = lax.rsqrt(var + EPS)
        o_ref[...] = xc * inv * gam_ref[...] + bet_ref[...]

    in_specs = [
        pl.BlockSpec((br, h), lambda p: (p, 0)),
        pl.BlockSpec((br, 1), lambda p: (p, 0)),
        pl.BlockSpec((br, h), lambda p: (0, 0)),
        pl.BlockSpec((2, h), lambda p: (0, 0)),
        pl.BlockSpec((1, h), lambda p: (0, 0)),
        pl.BlockSpec((1, h), lambda p: (0, 0)),
    ]
    args = [gathered, ttf, pos_emb, type_emb, gamma2, beta2]
    aliases = {}
    if out_prev is not None:
        in_specs.append(pl.BlockSpec(memory_space=pl.ANY))
        args.append(out_prev)
        aliases = {6: 0}
    return pl.pallas_call(
        body,
        grid=(nblk,),
        in_specs=in_specs,
        out_specs=pl.BlockSpec((br, h), lambda p, _k=k, _nblk=nblk: (_k * _nblk + p, 0)),
        out_shape=jax.ShapeDtypeStruct((n, h), jnp.float32),
        input_output_aliases=aliases,
    )(*args)


def kernel(input_ids, token_type_ids, word_emb, pos_emb, type_emb, gamma, beta):
    b, s = input_ids.shape
    h = word_emb.shape[1]
    n = b * s
    ids_flat = input_ids.reshape(n).astype(jnp.int32)
    ttf = token_type_ids.reshape(n, 1).astype(jnp.int32)
    gamma2 = gamma.reshape(1, h)
    beta2 = beta.reshape(1, h)
    nslice = 2
    sb = b // nslice
    sl = sb * s
    gathers = [
        _sc_gather(word_emb, ids_flat[k * sl : (k + 1) * sl], chunk=64)
        for k in range(nslice)
    ]
    out = None
    for k in range(nslice):
        out = _tc_fuse_slice(
            gathers[k],
            ttf[k * sl : (k + 1) * sl],
            pos_emb,
            type_emb,
            gamma2,
            beta2,
            out,
            k,
            br=2048,
            n=n,
        )
    return out.reshape(b, s, h)


# speedup vs baseline: 1.2161x; 1.0199x over previous
import functools

import jax
import jax.numpy as jnp
from jax import lax
from jax.experimental import pallas as pl
from jax.experimental.pallas import tpu as pltpu
from jax.experimental.pallas import tpu_sc as plsc

EPS = 1e-12

NC = 2
NS = 16
NW = NC * NS


def _sc_gather(table, ids_flat, chunk, nbuf):
    n, h = ids_flat.shape[0], table.shape[1]
    rows_per_w = n // NW
    nchunk = rows_per_w // chunk
    mesh = plsc.VectorSubcoreMesh(
        core_axis_name="c", subcore_axis_name="s", num_cores=NC, num_subcores=NS
    )

    @functools.partial(
        pl.kernel,
        mesh=mesh,
        out_type=jax.ShapeDtypeStruct((n, h), jnp.float32),
        scratch_types=[
            pltpu.VMEM((rows_per_w,), jnp.int32),
            pltpu.VMEM((nbuf, chunk, h), jnp.float32),
            pltpu.SemaphoreType.DMA((nbuf,)),
            pltpu.SemaphoreType.DMA((nbuf,)),
        ],
    )
    def gather_kernel(ids_hbm, table_hbm, out_hbm, idx_v, rows_v, gsem, wsem):
        wid = lax.axis_index("s") * NC + lax.axis_index("c")
        base = wid * rows_per_w
        pltpu.sync_copy(ids_hbm.at[pl.ds(base, rows_per_w)], idx_v)

        def start_gather(c):
            pltpu.async_copy(
                table_hbm.at[idx_v.at[pl.ds(c * chunk, chunk)]],
                rows_v.at[c % nbuf],
                gsem.at[c % nbuf],
            )

        def wait_gather(c):
            pltpu.make_async_copy(
                table_hbm.at[idx_v.at[pl.ds(c * chunk, chunk)]],
                rows_v.at[c % nbuf],
                gsem.at[c % nbuf],
            ).wait()

        def start_writeback(c):
            pltpu.async_copy(
                rows_v.at[c % nbuf],
                out_hbm.at[pl.ds(base + c * chunk, chunk)],
                wsem.at[c % nbuf],
            )

        def wait_writeback(c):
            pltpu.make_async_copy(
                rows_v.at[c % nbuf],
                out_hbm.at[pl.ds(base + c * chunk, chunk)],
                wsem.at[c % nbuf],
            ).wait()

        for c in range(min(nbuf, nchunk)):
            start_gather(c)
        for c in range(nchunk):
            wait_gather(c)
            start_writeback(c)
            if c + nbuf < nchunk:
                wait_writeback(c)
                start_gather(c + nbuf)
        for c in range(max(0, nchunk - nbuf), nchunk):
            wait_writeback(c)

    return gather_kernel(ids_flat, table)


def _tc_fuse_slice(gathered, ttf, pos_emb, type_emb, gamma2, beta2, out_prev, k, br, n):
    seq, h = gathered.shape
    nblk = seq // br

    def body(g_ref, tt_ref, pos_ref, type_ref, gam_ref, bet_ref, *refs):
        o_ref = refs[-1]
        x = g_ref[...] + pos_ref[...]
        ttv = tt_ref[...].astype(jnp.float32)
        t0 = type_ref[0:1, :]
        t1 = type_ref[1:2, :]
        x = x + t0 + ttv * (t1 - t0)
        mean = jnp.mean(x, axis=-1, keepdims=True)
        xc = x - mean
        var = jnp.mean(xc * xc, axis=-1, keepdims=True)
        inv = lax.rsqrt(var + EPS)
        o_ref[...] = xc * inv * gam_ref[...] + bet_ref[...]

    in_specs = [
        pl.BlockSpec((br, h), lambda p: (p, 0)),
        pl.BlockSpec((br, 1), lambda p: (p, 0)),
        pl.BlockSpec((br, h), lambda p: (0, 0)),
        pl.BlockSpec((2, h), lambda p: (0, 0)),
        pl.BlockSpec((1, h), lambda p: (0, 0)),
        pl.BlockSpec((1, h), lambda p: (0, 0)),
    ]
    args = [gathered, ttf, pos_emb, type_emb, gamma2, beta2]
    aliases = {}
    if out_prev is not None:
        in_specs.append(pl.BlockSpec(memory_space=pl.ANY))
        args.append(out_prev)
        aliases = {6: 0}
    return pl.pallas_call(
        body,
        grid=(nblk,),
        in_specs=in_specs,
        out_specs=pl.BlockSpec((br, h), lambda p, _k=k, _nblk=nblk: (_k * _nblk + p, 0)),
        out_shape=jax.ShapeDtypeStruct((n, h), jnp.float32),
        input_output_aliases=aliases,
    )(*args)


def kernel(input_ids, token_type_ids, word_emb, pos_emb, type_emb, gamma, beta):
    b, s = input_ids.shape
    h = word_emb.shape[1]
    n = b * s
    ids_flat = input_ids.reshape(n).astype(jnp.int32)
    ttf = token_type_ids.reshape(n, 1).astype(jnp.int32)
    gamma2 = gamma.reshape(1, h)
    beta2 = beta.reshape(1, h)
    nslice = 2
    sb = b // nslice
    sl = sb * s
    gathers = [
        _sc_gather(word_emb, ids_flat[k * sl : (k + 1) * sl], chunk=32, nbuf=4)
        for k in range(nslice)
    ]
    out = None
    for k in range(nslice):
        out = _tc_fuse_slice(
            gathers[k],
            ttf[k * sl : (k + 1) * sl],
            pos_emb,
            type_emb,
            gamma2,
            beta2,
            out,
            k,
            br=2048,
            n=n,
        )
    return out.reshape(b, s, h)
